# baseline (speedup 1.0000x reference)
import functools

import jax
import jax.numpy as jnp
from jax import lax
from jax.experimental import pallas as pl
from jax.experimental.pallas import tpu as pltpu
from jax.experimental.pallas import tpu_sc as plsc

_N, _E = 10000, 160000
_EP = 163840
_BLKN, _BLKE = 2000, 4096
_NCC, _NSC = 2, 16
_NWK = _NCC * _NSC
_LANES = 128
_GROUND = 16

_f32 = jnp.float32


def _dot(a, b):
    return lax.dot_general(a, b, (((1,), (0,)), ((), ())),
                           preferred_element_type=_f32)


def _red32(x):
    a = ((x[:, 0:8] + x[:, 8:16]) + x[:, 16:24]) + x[:, 24:32]
    t = a[:, 0:4] + a[:, 4:8]
    t = t[:, 0:2] + t[:, 2:4]
    return t[:, 0:1] + t[:, 1:2]


def _ln_exact(x, g, b):
    mu = _red32(x) * (1.0 / 32.0)
    xc = x - mu
    var = _red32(xc * xc) * (1.0 / 32.0)
    return xc / jnp.sqrt(var + 1e-5) * g + b


def _mlp_body(mode, ln, blk, n_pieces):
    def body(*refs):
        xs = [refs[i][...] for i in range(n_pieces)]
        off = n_pieces
        if mode == "proc":
            g_ref = refs[off]
            off += 1
        w0 = refs[off][...]
        b0 = refs[off + 1][...]
        wmid, bmid = refs[off + 2], refs[off + 3]
        wlast, blast = refs[off + 4][...], refs[off + 5][...]
        off += 6
        if ln:
            gst, bst = refs[off], refs[off + 1]
            off += 2
        o_ref = refs[off]

        if mode == "edge":
            d = xs[1][:, 0:3] - xs[2][:, 0:3]
            nrm = jnp.sqrt((d[:, 0:1] * d[:, 0:1] + d[:, 2:3] * d[:, 2:3])
                           + d[:, 1:2] * d[:, 1:2])
            x0 = jnp.concatenate(
                [xs[0][:, 0:1], d, nrm, jnp.zeros((blk, 3), _f32)], axis=1)
        elif mode == "proc":
            gcol = jnp.broadcast_to(g_ref[...], (blk, 1))
            x0 = jnp.concatenate(xs + [gcol], axis=1)
            pad = w0.shape[0] - x0.shape[1]
            x0 = jnp.concatenate([x0, jnp.zeros((blk, pad), _f32)], axis=1)
        else:
            x0 = xs[0]
        x = jnp.maximum(_dot(x0, w0) + b0, 0.0)
        if ln:
            x = _ln_exact(x, gst[0], bst[0])

        def step(j, x):
            x = jnp.maximum(_dot(x, wmid[j]) + bmid[j], 0.0)
            if ln:
                x = _ln_exact(x, gst[j + 1], bst[j + 1])
            return x

        x = lax.fori_loop(0, 32, step, x)
        o_ref[...] = jnp.maximum(_dot(x, wlast) + blast, 0.0)
    return body


def _const_spec(w):
    nd = w.ndim
    return pl.BlockSpec(w.shape, lambda i, _nd=nd: (0,) * _nd)


def _mlp(pieces, weights, *, m, blk, mode="plain", ln=False):
    ins, in_specs = [], []
    for arr, boff in pieces:
        k = arr.shape[1]
        ins.append(arr)
        in_specs.append(pl.BlockSpec((blk, k), lambda i, _b=boff: (i + _b, 0)))
    for w in weights:
        ins.append(w)
        in_specs.append(_const_spec(w))
    dout = weights[-3].shape[1] if ln else weights[-2].shape[1]
    return pl.pallas_call(
        _mlp_body(mode, ln, blk, len(pieces)),
        grid=(m // blk,),
        in_specs=in_specs,
        out_specs=pl.BlockSpec((blk, dout), lambda i: (i, 0)),
        out_shape=jax.ShapeDtypeStruct((m, dout), _f32),
        compiler_params=pltpu.CompilerParams(
            dimension_semantics=("arbitrary",)),
    )(*ins)


def _sc_gather(table, idx2d):
    return table[idx2d.reshape(-1)]


def _sc_gather_pallas(table, idx2d):
    n_blocks = idx2d.shape[0]
    bsz = n_blocks * _LANES
    per_w = bsz // _NWK
    n_ch = per_w // _LANES
    nround = n_ch // _GROUND
    mesh = plsc.VectorSubcoreMesh(core_axis_name="c", subcore_axis_name="s")

    @functools.partial(
        pl.kernel, mesh=mesh,
        out_type=jax.ShapeDtypeStruct((bsz, 16), _f32),
        scratch_types=[pltpu.VMEM((n_ch, _LANES), jnp.int32),
                       pltpu.VMEM((_GROUND * _LANES, 16), _f32),
                       pltpu.SemaphoreType.DMA],
        compiler_params=pltpu.CompilerParams(use_tc_tiling_on_sc=False))
    def gk(tab_h, idx_h, out_h, idx_v, rows_v, sem):
        wid = lax.axis_index("s") * _NCC + lax.axis_index("c")
        pltpu.sync_copy(idx_h.at[pl.ds(wid * n_ch, n_ch)], idx_v)

        def rnd(r, carry):
            cps = [pltpu.async_copy(tab_h.at[idx_v.at[r * _GROUND + c]],
                                    rows_v.at[pl.ds(c * _LANES, _LANES)], sem)
                   for c in range(_GROUND)]
            for cp in cps:
                cp.wait()
            pltpu.sync_copy(
                rows_v,
                out_h.at[pl.ds(wid * per_w + r * _GROUND * _LANES,
                               _GROUND * _LANES)])
            return carry

        lax.fori_loop(0, nround, rnd, 0)

    return gk(table, idx2d)


def _stack(p, ln, kpad):
    w0 = p["W"][0]
    w0p = jnp.zeros((kpad, 32), _f32).at[:w0.shape[0]].set(w0)
    ws = [w0p, p["b"][0][None, :],
          jnp.stack(p["W"][1:33]), jnp.stack(p["b"][1:33])[:, None, :],
          p["W"][33], p["b"][33][None, :]]
    if ln:
        ws += [jnp.stack(p["g"])[:, None, :], jnp.stack(p["beta"])[:, None, :]]
    return ws


def kernel(nodes, edges, senders, receivers, globals_, params):
    i32 = jnp.int32
    g11 = globals_[None, :].astype(_f32)

    nodes_cat8 = jnp.zeros((_N, 8), _f32)
    nodes_cat8 = nodes_cat8.at[:, :6].set(nodes).at[:, 6].set(globals_[0])
    pos16 = jnp.zeros((_N, 16), _f32).at[:, :3].set(nodes[:, :3])
    edges8 = jnp.zeros((_EP, 8), _f32).at[:_E, 0].set(edges[:, 0])
    s_pad = jnp.zeros((_EP,), i32).at[:_E].set(senders.astype(i32))
    r_pad = jnp.zeros((_EP,), i32).at[:_E].set(receivers.astype(i32))
    idx_sr = jnp.concatenate([s_pad, r_pad]).reshape(-1, _LANES)

    wne = _stack(params["node_enc"], False, 8)
    wee = _stack(params["edge_enc"], False, 8)
    wep = _stack(params["edge_proc"], True, 64)
    wnp = _stack(params["node_proc"], True, 40)
    wdec = _stack(params["decoder"], False, 16)
    wdec[4] = jnp.zeros((32, 8), _f32).at[:, :3].set(wdec[4])
    wdec[5] = jnp.zeros((1, 8), _f32).at[:, :3].set(wdec[5])

    off_r = _EP // _BLKE

    ve = _mlp([(nodes_cat8, 0)], wne, m=_N, blk=_BLKN)
    posg = _sc_gather(pos16, idx_sr)
    ee = _mlp([(edges8, 0), (posg, 0), (posg, off_r)], wee,
              m=_EP, blk=_BLKE, mode="edge")

    for _ in range(2):
        veg = _sc_gather(ve, idx_sr)
        ee = _mlp([(ee, 0), (veg, 0), (veg, off_r)], [g11] + wep,
                  m=_EP, blk=_BLKE, mode="proc", ln=True)
        scat = jax.ops.segment_sum(ee[:_E], receivers, num_segments=_N)
        ve = _mlp([(scat, 0), (ve, 0)], [g11] + wnp,
                  m=_N, blk=_BLKN, mode="proc", ln=True)

    out8 = _mlp([(ve, 0)], wdec, m=_N, blk=_BLKN)
    return out8[:, :3]


# speedup vs baseline: 2.6039x; 2.6039x over previous
import functools

import jax
import jax.numpy as jnp
from jax import lax
from jax.experimental import pallas as pl
from jax.experimental.pallas import tpu as pltpu
from jax.experimental.pallas import tpu_sc as plsc

_N, _E = 10000, 160000
_EP = 163840
_BLKN, _BLKE = 10000, 4096
_NCC, _NSC = 2, 16
_NWK = _NCC * _NSC
_LANES = 128
_GROUND = 16

_f32 = jnp.float32


def _dot(a, b):
    return lax.dot_general(a, b, (((1,), (0,)), ((), ())),
                           preferred_element_type=_f32)


def _red32_packed(x):
    outs = []
    for o in (0, 32, 64, 96):
        a = ((x[:, o:o + 8] + x[:, o + 8:o + 16]) + x[:, o + 16:o + 24]) \
            + x[:, o + 24:o + 32]
        t = a[:, 0:4] + a[:, 4:8]
        t = t[:, 0:2] + t[:, 2:4]
        outs.append(t[:, 0:1] + t[:, 1:2])
    return outs


def _bcast32(cols):
    r = cols[0].shape[0]
    return jnp.concatenate(
        [jnp.broadcast_to(c, (r, 32)) for c in cols], axis=1)


def _ln_packed(x, g, b):
    mu = _bcast32([s * (1.0 / 32.0) for s in _red32_packed(x)])
    xc = x - mu
    var = _bcast32([v * (1.0 / 32.0) for v in _red32_packed(xc * xc)])
    return xc / jnp.sqrt(var + 1e-5) * g + b


def _mlp_body(mode, ln, blk, n_pieces):
    rpk = blk // 4

    def body(*refs):
        xs = [refs[i][...] for i in range(n_pieces)]
        off = n_pieces
        if mode == "proc":
            g_ref = refs[off]
            off += 1
        w0 = refs[off][...]
        b0 = refs[off + 1][...]
        wmid, bmid = refs[off + 2], refs[off + 3]
        wlast, blast = refs[off + 4][...], refs[off + 5][...]
        off += 6
        if ln:
            gst, bst = refs[off], refs[off + 1]
            off += 2
        o_ref = refs[off]

        if mode == "edge":
            d = xs[1] - xs[2]
            chunks = []
            for k in range(4):
                dk = d[:, 16 * k:16 * k + 3]
                nrm = jnp.sqrt((dk[:, 0:1] * dk[:, 0:1]
                                + dk[:, 2:3] * dk[:, 2:3])
                               + dk[:, 1:2] * dk[:, 1:2])
                chunks += [xs[0][:, 8 * k:8 * k + 1], dk, nrm,
                           jnp.zeros((rpk, 3), _f32)]
            x0 = jnp.concatenate(chunks, axis=1)
        elif mode == "proc":
            gcol = jnp.broadcast_to(g_ref[...], (rpk, 1))
            kin = xs[0].shape[1] // 4
            zpad = w0.shape[0] // 4 - n_pieces * kin - 1
            chunks = []
            for k in range(4):
                for xv in xs:
                    chunks.append(xv[:, kin * k:kin * (k + 1)])
                chunks += [gcol, jnp.zeros((rpk, zpad), _f32)]
            x0 = jnp.concatenate(chunks, axis=1)
        else:
            x0 = xs[0]
        x = jnp.maximum(_dot(x0, w0) + b0, 0.0)
        if ln:
            x = _ln_packed(x, gst[0], bst[0])

        def step(j, x):
            x = jnp.maximum(_dot(x, wmid[j]) + bmid[j], 0.0)
            if ln:
                x = _ln_packed(x, gst[j + 1], bst[j + 1])
            return x

        x = lax.fori_loop(0, 32, step, x)
        o_ref[...] = jnp.maximum(_dot(x, wlast) + blast, 0.0)
    return body


def _const_spec(w):
    nd = w.ndim
    return pl.BlockSpec(w.shape, lambda i, _nd=nd: (0,) * _nd)


def _mlp(pieces, weights, *, m, blk, mode="plain", ln=False):
    ins, in_specs = [], []
    for arr, boff in pieces:
        k = arr.shape[1]
        ins.append(arr)
        in_specs.append(
            pl.BlockSpec((blk // 4, k), lambda i, _b=boff: (i + _b, 0)))
    for w in weights:
        ins.append(w)
        in_specs.append(_const_spec(w))
    dout4 = weights[-3].shape[1] if ln else weights[-2].shape[1]
    return pl.pallas_call(
        _mlp_body(mode, ln, blk, len(pieces)),
        grid=(m // blk,),
        in_specs=in_specs,
        out_specs=pl.BlockSpec((blk // 4, dout4), lambda i: (i, 0)),
        out_shape=jax.ShapeDtypeStruct((m // 4, dout4), _f32),
        compiler_params=pltpu.CompilerParams(
            dimension_semantics=("arbitrary",)),
    )(*ins)


def _sc_gather(table, idx2d):
    n_blocks = idx2d.shape[0]
    bsz = n_blocks * _LANES
    per_w = bsz // _NWK
    n_ch = per_w // _LANES
    nround = n_ch // _GROUND
    mesh = plsc.VectorSubcoreMesh(core_axis_name="c", subcore_axis_name="s")

    @functools.partial(
        pl.kernel, mesh=mesh,
        out_type=jax.ShapeDtypeStruct((bsz, 16), _f32),
        scratch_types=[pltpu.VMEM((n_ch, _LANES), jnp.int32),
                       pltpu.VMEM((_GROUND * _LANES, 16), _f32),
                       pltpu.SemaphoreType.DMA],
        compiler_params=pltpu.CompilerParams(use_tc_tiling_on_sc=False))
    def gk(tab_h, idx_h, out_h, idx_v, rows_v, sem):
        wid = lax.axis_index("s") * _NCC + lax.axis_index("c")
        pltpu.sync_copy(idx_h.at[pl.ds(wid * n_ch, n_ch)], idx_v)

        def rnd(r, carry):
            cps = [pltpu.async_copy(tab_h.at[idx_v.at[r * _GROUND + c]],
                                    rows_v.at[pl.ds(c * _LANES, _LANES)], sem)
                   for c in range(_GROUND)]
            for cp in cps:
                cp.wait()
            pltpu.sync_copy(
                rows_v,
                out_h.at[pl.ds(wid * per_w + r * _GROUND * _LANES,
                               _GROUND * _LANES)])
            return carry

        lax.fori_loop(0, nround, rnd, 0)

    return gk(table, idx2d)


def _stack(p, ln, kpad, dout_pad=None):
    eye4 = jnp.eye(4, dtype=_f32)
    w0 = p["W"][0]
    w0p = jnp.zeros((kpad, 32), _f32).at[:w0.shape[0]].set(w0)
    wmid = jnp.stack([jnp.kron(eye4, w) for w in p["W"][1:33]])
    bmid = jnp.stack([jnp.tile(b, 4) for b in p["b"][1:33]])[:, None, :]
    wlast, blast = p["W"][33], p["b"][33]
    if dout_pad is not None:
        wlast = jnp.zeros((32, dout_pad), _f32).at[:, :wlast.shape[1]].set(wlast)
        blast = jnp.zeros((dout_pad,), _f32).at[:blast.shape[0]].set(blast)
    ws = [jnp.kron(eye4, w0p), jnp.tile(p["b"][0], 4)[None, :],
          wmid, bmid,
          jnp.kron(eye4, wlast), jnp.tile(blast, 4)[None, :]]
    if ln:
        ws += [jnp.stack([jnp.tile(g, 4) for g in p["g"]])[:, None, :],
               jnp.stack([jnp.tile(b, 4) for b in p["beta"]])[:, None, :]]
    return ws


def kernel(nodes, edges, senders, receivers, globals_, params):
    i32 = jnp.int32
    g11 = globals_[None, :].astype(_f32)

    nodes_cat8 = jnp.zeros((_N, 8), _f32)
    nodes_cat8 = nodes_cat8.at[:, :6].set(nodes).at[:, 6].set(globals_[0])
    pos16 = jnp.zeros((_N, 16), _f32).at[:, :3].set(nodes[:, :3])
    edges8 = jnp.zeros((_EP, 8), _f32).at[:_E, 0].set(edges[:, 0])
    s_pad = jnp.zeros((_EP,), i32).at[:_E].set(senders.astype(i32))
    r_pad = jnp.zeros((_EP,), i32).at[:_E].set(receivers.astype(i32))
    idx_sr = jnp.concatenate([s_pad, r_pad]).reshape(-1, _LANES)

    wne = _stack(params["node_enc"], False, 8)
    wee = _stack(params["edge_enc"], False, 8)
    wep = _stack(params["edge_proc"], True, 64)
    wnp = _stack(params["node_proc"], True, 64)
    wdec = _stack(params["decoder"], False, 16, dout_pad=8)

    off_r = _EP // _BLKE

    vep = _mlp([(nodes_cat8.reshape(_N // 4, 32), 0)], wne, m=_N, blk=_BLKN)
    posg = _sc_gather(pos16, idx_sr)
    eep = _mlp([(edges8.reshape(_EP // 4, 32), 0),
                (posg.reshape(_EP // 2, 64), 0),
                (posg.reshape(_EP // 2, 64), off_r)], wee,
               m=_EP, blk=_BLKE, mode="edge")

    for _ in range(2):
        veg = _sc_gather(vep.reshape(_N, 16), idx_sr)
        vegp = veg.reshape(_EP // 2, 64)
        eep = _mlp([(eep, 0), (vegp, 0), (vegp, off_r)], [g11] + wep,
                   m=_EP, blk=_BLKE, mode="proc", ln=True)
        scat = jax.ops.segment_sum(eep.reshape(_EP, 16)[:_E], receivers,
                                   num_segments=_N)
        vep = _mlp([(scat.reshape(_N // 4, 64), 0), (vep, 0)], [g11] + wnp,
                   m=_N, blk=_BLKN, mode="proc", ln=True)

    out = _mlp([(vep, 0)], wdec, m=_N, blk=_BLKN)
    return out.reshape(_N, 8)[:, :3]
